# baseline (speedup 1.0000x reference)
import functools

import jax
import jax.numpy as jnp
from jax import lax
from jax.experimental import pallas as pl
from jax.experimental.pallas import tpu as pltpu
from jax.experimental.pallas import tpu_sc as plsc

_S = 1024
_D = 128
_T = 2 * _S - 1
_R_SC = 352
_BI = 8


def _sc_part(table):
    mesh = plsc.VectorSubcoreMesh(core_axis_name="c", subcore_axis_name="s")
    rows_per_w = _R_SC // 32

    @functools.partial(
        pl.kernel,
        out_type=jax.ShapeDtypeStruct((_R_SC, _S, _D), jnp.float32),
        mesh=mesh,
        scratch_types=[pltpu.MemorySpace.VMEM_SHARED((_T, _D), jnp.float32)],
    )
    def run(table_hbm, out_hbm, shared):
        cid = lax.axis_index("c")
        sid = lax.axis_index("s")

        @pl.when(sid == 0)
        def _stage():
            pltpu.sync_copy(table_hbm, shared)

        plsc.subcore_barrier()

        wid = sid * 2 + cid
        base = wid * rows_per_w
        for r in range(rows_per_w):
            i = base + r
            start = (_S - 1) - i
            pltpu.sync_copy(shared.at[pl.ds(start, _S)], out_hbm.at[i])

    return run(table)


def _tc_body(table_ref, out_ref):
    i0 = _R_SC + pl.program_id(0) * _BI
    for k in range(_BI):
        start = (_S - 1) - (i0 + k)
        out_ref[k] = table_ref[pl.ds(start, _S), :]


def _tc_part(table):
    n = _S - _R_SC
    return pl.pallas_call(
        _tc_body,
        grid=(n // _BI,),
        in_specs=[pl.BlockSpec((_T, _D), lambda i: (0, 0))],
        out_specs=pl.BlockSpec((_BI, _S, _D), lambda i: (i, 0, 0)),
        out_shape=jax.ShapeDtypeStruct((n, _S, _D), jnp.float32),
    )(table)


def kernel(x, table):
    del x
    return jnp.concatenate([_sc_part(table), _tc_part(table)], axis=0)


# speedup vs baseline: 2.0407x; 2.0407x over previous
import jax
import jax.numpy as jnp
from jax.experimental import pallas as pl
from jax.experimental.pallas import tpu as pltpu

_S = 1024
_D = 128
_T = 2 * _S - 1
_BI = 8


def _body(table_ref, out_ref, sem):
    i0 = pl.program_id(0) * _BI
    copies = []
    for k in range(_BI):
        i = i0 + k
        start = (_S - 1) - i
        cp = pltpu.make_async_copy(
            table_ref.at[pl.ds(start, _S), :], out_ref.at[i], sem
        )
        cp.start()
        copies.append(cp)
    for cp in copies:
        cp.wait()


def kernel(x, table):
    del x
    return pl.pallas_call(
        _body,
        grid=(_S // _BI,),
        in_specs=[pl.BlockSpec((_T, _D), lambda i: (0, 0))],
        out_specs=pl.BlockSpec(memory_space=pltpu.MemorySpace.HBM),
        out_shape=jax.ShapeDtypeStruct((_S, _S, _D), jnp.float32),
        scratch_shapes=[pltpu.SemaphoreType.DMA],
    )(table)


# speedup vs baseline: 3.1756x; 1.5562x over previous
import jax
import jax.numpy as jnp
from jax.experimental import pallas as pl
from jax.experimental.pallas import tpu as pltpu

_MAX_LEN = 1024
_D = 128
_BI = 16


def _body(table_ref, out_ref):
    i0 = pl.program_id(0) * _BI
    for k in range(_BI):
        start = (_MAX_LEN - 1) - (i0 + k)
        out_ref[k] = table_ref[pl.ds(start, _MAX_LEN), :]


def kernel(x, table):
    del x
    s = _MAX_LEN
    return pl.pallas_call(
        _body,
        grid=(s // _BI,),
        in_specs=[
            pl.BlockSpec((2 * s - 1, _D), lambda i: (0, 0)),
        ],
        out_specs=pl.BlockSpec((_BI, s, _D), lambda i: (i, 0, 0)),
        out_shape=jax.ShapeDtypeStruct((s, s, _D), jnp.float32),
    )(table)
